# baseline (speedup 1.0000x reference)
import functools

import jax
import jax.numpy as jnp
from jax import lax
from jax.experimental import pallas as pl
from jax.experimental.pallas import tpu as pltpu
from jax.experimental.pallas import tpu_sc as plsc

VOCAB = 1000000
DIM = 64
B = 4096
L = 200
N = B * L
NC = 2
NS = 16
NW = NC * NS
BT = B // 128
UNITS = L * BT
U_PER_W = UNITS // NW
NBUF = 4

_mesh = plsc.VectorSubcoreMesh(core_axis_name="c", subcore_axis_name="s")


@functools.partial(
    pl.kernel,
    mesh=_mesh,
    out_type=jax.ShapeDtypeStruct((L, DIM // 8, BT, 8, 128), jnp.float32),
    scratch_types=[
        pltpu.VMEM((U_PER_W, 128), jnp.int32),
        [pltpu.VMEM((128, DIM), jnp.float32)] * NBUF,
        [pltpu.VMEM((DIM // 8, 8, 128), jnp.float32)] * NBUF,
        [pltpu.SemaphoreType.DMA] * NBUF,
        [pltpu.SemaphoreType.DMA] * NBUF,
    ],
    compiler_params=pltpu.CompilerParams(
        use_tc_tiling_on_sc=False, needs_layout_passes=False),
)
def _emb_lookup(tokens_hbm, weight_hbm, out_hbm, idx_v, rows, trans,
                gsem, ssem):
    wid = lax.axis_index("s") * NC + lax.axis_index("c")
    pltpu.sync_copy(tokens_hbm.at[wid], idx_v)
    ubase = wid * U_PER_W
    jb4 = [(lax.iota(jnp.int32, 16) + 16 * k) // 8 for k in range(4)]
    jj4 = [(lax.iota(jnp.int32, 16) + 16 * k) % 8 for k in range(4)]

    def unit_lt(u):
        k = ubase + u
        return k // BT, k % BT

    def fire_gather(b, u):
        pltpu.async_copy(weight_hbm.at[idx_v.at[u]], rows[b], gsem[b])

    def wait_gather(b):
        pltpu.make_async_copy(
            weight_hbm.at[idx_v.at[0]], rows[b], gsem[b]).wait()

    def fire_store(b, u):
        l, tb = unit_lt(u)
        pltpu.async_copy(trans[b], out_hbm.at[l, :, tb], ssem[b])

    def wait_store(b, u):
        l, tb = unit_lt(u)
        pltpu.make_async_copy(
            trans[b], out_hbm.at[l, :, tb], ssem[b]).wait()

    def transpose(b):
        @plsc.parallel_loop(0, 128, 1, unroll=8)
        def _(r):
            rv = lax.broadcast(r, (16,))
            for k in range(4):
                v = rows[b][r, pl.ds(16 * k, 16)]
                plsc.store_scatter(trans[b], [jb4[k], jj4[k], rv], v)

    for b in range(NBUF):
        fire_gather(b, b)

    @pl.loop(0, U_PER_W, step=NBUF)
    def _(g):
        for b in range(NBUF):
            u = g + b
            wait_gather(b)

            @pl.when(g >= NBUF)
            def _():
                wait_store(b, u - NBUF)

            transpose(b)
            fire_store(b, u)

            @pl.when(u + NBUF < U_PER_W)
            def _():
                fire_gather(b, u + NBUF)

    for b in range(NBUF):
        wait_store(b, U_PER_W - NBUF + b)


def kernel(tokens, weight):
    toku = tokens.T.reshape(NW, U_PER_W, 128).astype(jnp.int32)
    out5 = _emb_lookup(toku, weight)
    return out5.transpose(2, 4, 0, 1, 3).reshape(B, L, DIM)


# speedup vs baseline: 1.0595x; 1.0595x over previous
import functools

import jax
import jax.numpy as jnp
from jax import lax
from jax.experimental import pallas as pl
from jax.experimental.pallas import tpu as pltpu
from jax.experimental.pallas import tpu_sc as plsc

VOCAB = 1000000
DIM = 64
B = 4096
L = 200
N = B * L
NC = 2
NS = 16
NW = NC * NS
N_PER_W = N // NW
CHUNK = 128
NCHUNK = N_PER_W // CHUNK
NBUF = 4

_mesh = plsc.VectorSubcoreMesh(core_axis_name="c", subcore_axis_name="s")


@functools.partial(
    pl.kernel,
    mesh=_mesh,
    out_type=jax.ShapeDtypeStruct((NW, N_PER_W, DIM), jnp.float32),
    scratch_types=[
        pltpu.VMEM((NCHUNK, CHUNK), jnp.int32),
        [pltpu.VMEM((CHUNK, DIM), jnp.float32)] * NBUF,
        [pltpu.SemaphoreType.DMA] * NBUF,
        [pltpu.SemaphoreType.DMA] * NBUF,
    ],
    compiler_params=pltpu.CompilerParams(use_tc_tiling_on_sc=False),
)
def _emb_lookup(tokens_hbm, weight_hbm, out_hbm, idx_v, rows, gsem, ssem):
    wid = lax.axis_index("s") * NC + lax.axis_index("c")
    pltpu.sync_copy(tokens_hbm.at[wid], idx_v)

    def fire_gather(b, c):
        pltpu.async_copy(weight_hbm.at[idx_v.at[c]], rows[b], gsem[b])

    def fire_store(b, c):
        pltpu.async_copy(
            rows[b], out_hbm.at[wid, pl.ds(c * CHUNK, CHUNK)], ssem[b])

    def wait_gather(b):
        pltpu.make_async_copy(weight_hbm.at[idx_v.at[0]], rows[b], gsem[b]).wait()

    def wait_store(b, c):
        pltpu.make_async_copy(
            rows[b], out_hbm.at[wid, pl.ds(c * CHUNK, CHUNK)], ssem[b]).wait()

    for b in range(NBUF):
        fire_gather(b, b)

    @pl.loop(0, NCHUNK, step=NBUF)
    def _(g):
        for b in range(NBUF):
            wait_gather(b)
            fire_store(b, g + b)
        for b in range(NBUF):
            @pl.when(g + b + NBUF < NCHUNK)
            def _():
                wait_store(b, g + b)
                fire_gather(b, g + b + NBUF)

    for b in range(NBUF):
        wait_store(b, NCHUNK - NBUF + b)


def kernel(tokens, weight):
    tokens3d = tokens.reshape(NW, NCHUNK, CHUNK).astype(jnp.int32)
    out = _emb_lookup(tokens3d, weight)
    out2 = lax.optimization_barrier(out.reshape(N // 2, 2 * DIM))
    return out2.reshape(B, L, DIM)
